# baseline (speedup 1.0000x reference)
import functools

import jax
import jax.numpy as jnp
from jax import lax
from jax.experimental import pallas as pl
from jax.experimental.pallas import tpu as pltpu
from jax.experimental.pallas import tpu_sc as plsc

N = 10000
D = 128
G = 64

NC, NS = 2, 16
NW = NC * NS
CH = 128
SUP = 8
TSUP = 320
EP = TSUP * SUP * CH
K0, K1 = 20, 0
NP = 10240
BN = 2000
NB = N // BN


def _sc_body(x_hbm, eidx_hbm, zero_hbm, out_hbm,
             acc_sh, idxv, rows_a, rows_b, sem_a, sem_b):
    rows = (rows_a, rows_b)
    sems = (sem_a, sem_b)
    cid = lax.axis_index("c")
    sid = lax.axis_index("s")
    wid = sid * NC + cid

    with jax.named_scope("zero_phase"):
        pltpu.sync_copy(zero_hbm, rows_a)
        for k in range(NP // (NS * CH)):
            r0 = (sid * (NP // (NS * CH)) + k) * CH
            pltpu.sync_copy(rows_a, acc_sh.at[pl.ds(r0, CH)])
        plsc.subcore_barrier()

    def _wait(buf, sem):
        pltpu.make_async_copy(zero_hbm, buf, sem).wait()

    nsup = jnp.where(cid == 0, K0, K1)
    base = jnp.where(cid == 0, sid * K0, NS * K0 + sid * K1)

    with jax.named_scope("edge_phase"):
        @pl.loop(0, nsup)
        def _(s):
            rb = (base + s) * (2 * SUP)
            pltpu.sync_copy(eidx_hbm.at[pl.ds(rb, 2 * SUP)], idxv)
            pltpu.async_copy(x_hbm.at[idxv.at[0]], rows[0], sems[0])
            for c in range(SUP):
                b = c % 2
                _wait(rows[b], sems[b])
                if c + 1 < SUP:
                    b2 = (c + 1) % 2
                    pltpu.async_copy(x_hbm.at[idxv.at[c + 1]], rows[b2], sems[b2])
                pltpu.sync_copy(rows[b], acc_sh.at[idxv.at[SUP + c]], add=True)

        plsc.subcore_barrier()

    with jax.named_scope("readout_phase"):
        for k in range(NP // (NS * CH)):
            r0 = (sid * (NP // (NS * CH)) + k) * CH
            pltpu.sync_copy(acc_sh.at[pl.ds(r0, CH)], rows_a)
            pltpu.sync_copy(rows_a, out_hbm.at[cid].at[pl.ds(r0, CH)])


_sc_aggregate = functools.partial(
    pl.kernel,
    out_type=jax.ShapeDtypeStruct((NC, NP, D), jnp.float32),
    mesh=plsc.VectorSubcoreMesh(core_axis_name="c", subcore_axis_name="s"),
    scratch_types=[
        pltpu.VMEM_SHARED((NP, D), jnp.float32),
        pltpu.VMEM((2 * SUP, CH), jnp.int32),
        pltpu.VMEM((CH, D), jnp.float32),
        pltpu.VMEM((CH, D), jnp.float32),
        pltpu.SemaphoreType.DMA,
        pltpu.SemaphoreType.DMA,
    ],
)(_sc_body)


def _tc_body(aggp_ref, x_ref, bid_ref, wn_ref, ws_ref, b_ref, out_ref):
    i = pl.program_id(0)

    @pl.when(i == 0)
    def _():
        out_ref[...] = jnp.zeros_like(out_ref)

    agg = aggp_ref[0] + aggp_ref[1]
    h = jnp.dot(agg, wn_ref[...], preferred_element_type=jnp.float32)
    h += jnp.dot(x_ref[...], ws_ref[...], preferred_element_type=jnp.float32)
    h = jnp.maximum(h + b_ref[...], 0.0)
    bid = bid_ref[0, 0, :]
    gids = lax.broadcasted_iota(jnp.int32, (G, BN), 0)
    onehot = (gids == bid[None, :]).astype(jnp.float32)
    out_ref[...] += jnp.dot(onehot, h, preferred_element_type=jnp.float32)


def _tc_finish(aggp, x, bids3, w_nbr, w_self, b2):
    return pl.pallas_call(
        _tc_body,
        grid=(NB,),
        in_specs=[
            pl.BlockSpec((NC, BN, D), lambda i: (0, i, 0)),
            pl.BlockSpec((BN, D), lambda i: (i, 0)),
            pl.BlockSpec((1, 1, BN), lambda i: (i, 0, 0)),
            pl.BlockSpec((D, D), lambda i: (0, 0)),
            pl.BlockSpec((D, D), lambda i: (0, 0)),
            pl.BlockSpec((1, D), lambda i: (0, 0)),
        ],
        out_specs=pl.BlockSpec((G, D), lambda i: (0, 0)),
        out_shape=jax.ShapeDtypeStruct((G, D), jnp.float32),
    )(aggp, x, bids3, w_nbr, w_self, b2)


def kernel(x, edge_index, batch_ids, W_nbr, W_self, b):
    E = edge_index.shape[1]
    src4 = jnp.pad(edge_index[0], (0, EP - E)).reshape(TSUP, SUP, CH)
    dst4 = jnp.pad(edge_index[1], (0, EP - E),
                   constant_values=N).reshape(TSUP, SUP, CH)
    eidx = jnp.concatenate([src4, dst4], axis=1).reshape(TSUP * 2 * SUP, CH)
    zeros = jnp.zeros((CH, D), jnp.float32)
    aggp = _sc_aggregate(x, eidx, zeros)
    bids3 = batch_ids.reshape(NB, 1, BN)
    return _tc_finish(aggp, x, bids3, W_nbr, W_self, b.reshape(1, D))


# speedup vs baseline: 1.6370x; 1.6370x over previous
import functools

import jax
import jax.numpy as jnp
from jax import lax
from jax.experimental import pallas as pl
from jax.experimental.pallas import tpu as pltpu
from jax.experimental.pallas import tpu_sc as plsc

N = 10000
D = 128
G = 64

NC, NS = 2, 16
CH = 128
SUP = 4
TSUP = 640
K0, K1 = 36, 4
EP = TSUP * SUP * CH
NZC = 78
BN = 2000
NB = N // BN


def _sc_body(x_hbm, eidx_hbm, zero_hbm, out_hbm,
             acc_sh, idxv, rows_a, rows_b, rows_c, sem_a, sem_b, sem_c):
    rows = (rows_a, rows_b, rows_c)
    sems = (sem_a, sem_b, sem_c)
    cid = lax.axis_index("c")
    sid = lax.axis_index("s")

    with jax.named_scope("zero_phase"):
        pltpu.sync_copy(zero_hbm, rows_a)
        for k in range(5):
            cno = sid * 5 + k

            @pl.when(cno < NZC)
            def _():
                pltpu.sync_copy(rows_a, acc_sh.at[pl.ds(cno * CH, CH)])

        @pl.when(sid == NS - 1)
        def _():
            pltpu.sync_copy(rows_a.at[pl.ds(0, 16)],
                            acc_sh.at[pl.ds(NZC * CH, 16)])

        plsc.subcore_barrier()

    nsup = jnp.where(cid == 0, K0, K1)
    base = jnp.where(cid == 0, sid * K0, NS * K0 + sid * K1)

    def _wait(buf, sem):
        pltpu.make_async_copy(zero_hbm, buf, sem).wait()

    with jax.named_scope("edge_phase"):
        @pl.loop(0, nsup)
        def _(s):
            rb = (base + s) * (2 * SUP)
            pltpu.sync_copy(eidx_hbm.at[pl.ds(rb, 2 * SUP)], idxv)
            pltpu.async_copy(x_hbm.at[idxv.at[0]], rows[0], sems[0])
            pltpu.async_copy(x_hbm.at[idxv.at[1]], rows[1], sems[1])
            for c in range(SUP):
                b = c % 3
                _wait(rows[b], sems[b])
                if c + 2 < SUP:
                    b2 = (c + 2) % 3
                    pltpu.async_copy(x_hbm.at[idxv.at[c + 2]], rows[b2], sems[b2])
                pltpu.sync_copy(rows[b], acc_sh.at[idxv.at[SUP + c]], add=True)

        plsc.subcore_barrier()

    with jax.named_scope("readout_phase"):
        for k in range(5):
            cno = sid * 5 + k

            @pl.when(cno < NZC)
            def _():
                pltpu.sync_copy(acc_sh.at[pl.ds(cno * CH, CH)], rows_a)
                pltpu.sync_copy(rows_a, out_hbm.at[cid].at[pl.ds(cno * CH, CH)])

        @pl.when(sid == NS - 1)
        def _():
            pltpu.sync_copy(acc_sh.at[pl.ds(NZC * CH, 16)],
                            rows_b.at[pl.ds(0, 16)])
            pltpu.sync_copy(rows_b.at[pl.ds(0, 16)],
                            out_hbm.at[cid].at[pl.ds(NZC * CH, 16)])


_sc_aggregate = functools.partial(
    pl.kernel,
    out_type=jax.ShapeDtypeStruct((NC, N, D), jnp.float32),
    mesh=plsc.VectorSubcoreMesh(core_axis_name="c", subcore_axis_name="s"),
    scratch_types=[
        pltpu.VMEM_SHARED((N, D), jnp.float32),
        pltpu.VMEM((2 * SUP, CH), jnp.int32),
        pltpu.VMEM((CH, D), jnp.float32),
        pltpu.VMEM((CH, D), jnp.float32),
        pltpu.VMEM((CH, D), jnp.float32),
        pltpu.SemaphoreType.DMA,
        pltpu.SemaphoreType.DMA,
        pltpu.SemaphoreType.DMA,
    ],
)(_sc_body)


def _tc_body(aggp_ref, x_ref, bid_ref, wn_ref, ws_ref, b_ref, out_ref):
    i = pl.program_id(0)

    @pl.when(i == 0)
    def _():
        out_ref[...] = jnp.zeros_like(out_ref)

    agg = aggp_ref[0] + aggp_ref[1]
    h = jnp.dot(agg, wn_ref[...], preferred_element_type=jnp.float32)
    h += jnp.dot(x_ref[...], ws_ref[...], preferred_element_type=jnp.float32)
    h = jnp.maximum(h + b_ref[...], 0.0)
    bid = bid_ref[0, 0, :]
    gids = lax.broadcasted_iota(jnp.int32, (G, BN), 0)
    onehot = (gids == bid[None, :]).astype(jnp.float32)
    out_ref[...] += jnp.dot(onehot, h, preferred_element_type=jnp.float32)


def _tc_finish(aggp, x, bids3, w_nbr, w_self, b2):
    return pl.pallas_call(
        _tc_body,
        grid=(NB,),
        in_specs=[
            pl.BlockSpec((NC, BN, D), lambda i: (0, i, 0)),
            pl.BlockSpec((BN, D), lambda i: (i, 0)),
            pl.BlockSpec((1, 1, BN), lambda i: (i, 0, 0)),
            pl.BlockSpec((D, D), lambda i: (0, 0)),
            pl.BlockSpec((D, D), lambda i: (0, 0)),
            pl.BlockSpec((1, D), lambda i: (0, 0)),
        ],
        out_specs=pl.BlockSpec((G, D), lambda i: (0, 0)),
        out_shape=jax.ShapeDtypeStruct((G, D), jnp.float32),
    )(aggp, x, bids3, w_nbr, w_self, b2)


def kernel(x, edge_index, batch_ids, W_nbr, W_self, b):
    E = edge_index.shape[1]
    xz = jnp.concatenate([x, jnp.zeros((1, D), jnp.float32)], axis=0)
    src3 = jnp.pad(edge_index[0], (0, EP - E),
                   constant_values=N).reshape(TSUP, SUP, CH)
    dst3 = jnp.pad(edge_index[1], (0, EP - E)).reshape(TSUP, SUP, CH)
    eidx = jnp.concatenate([src3, dst3], axis=1).reshape(TSUP * 2 * SUP, CH)
    zeros = jnp.zeros((CH, D), jnp.float32)
    aggp = _sc_aggregate(xz, eidx, zeros)
    bids3 = batch_ids.reshape(NB, 1, BN)
    return _tc_finish(aggp, x, bids3, W_nbr, W_self, b.reshape(1, D))


# speedup vs baseline: 1.6508x; 1.0084x over previous
import functools

import jax
import jax.numpy as jnp
from jax import lax
from jax.experimental import pallas as pl
from jax.experimental.pallas import tpu as pltpu
from jax.experimental.pallas import tpu_sc as plsc

N = 10000
D = 128
G = 64

NC, NS = 2, 16
CH = 128
SUP = 4
TSUP = 640
K0, K1 = 37, 3
EP = TSUP * SUP * CH
NZC = 78
BN = 2000
NB = N // BN


def _sc_body(x_hbm, eidx_hbm, zero_hbm, out_hbm,
             acc_sh, idxv, rows_a, rows_b, rows_c, sem_a, sem_b, sem_c):
    rows = (rows_a, rows_b, rows_c)
    sems = (sem_a, sem_b, sem_c)
    cid = lax.axis_index("c")
    sid = lax.axis_index("s")

    with jax.named_scope("zero_phase"):
        pltpu.sync_copy(zero_hbm, rows_a)
        for k in range(5):
            cno = sid * 5 + k

            @pl.when(cno < NZC)
            def _():
                pltpu.sync_copy(rows_a, acc_sh.at[pl.ds(cno * CH, CH)])

        @pl.when(sid == NS - 1)
        def _():
            pltpu.sync_copy(rows_a.at[pl.ds(0, 16)],
                            acc_sh.at[pl.ds(NZC * CH, 16)])

        plsc.subcore_barrier()

    nsup = jnp.where(cid == 0, K0, K1)
    base = jnp.where(cid == 0, sid * K0, NS * K0 + sid * K1)

    def _wait(buf, sem):
        pltpu.make_async_copy(zero_hbm, buf, sem).wait()

    with jax.named_scope("edge_phase"):
        @pl.loop(0, nsup)
        def _(s):
            rb = (base + s) * (2 * SUP)
            pltpu.sync_copy(eidx_hbm.at[pl.ds(rb, 2 * SUP)], idxv)
            pltpu.async_copy(x_hbm.at[idxv.at[0]], rows[0], sems[0])
            pltpu.async_copy(x_hbm.at[idxv.at[1]], rows[1], sems[1])
            for c in range(SUP):
                b = c % 3
                _wait(rows[b], sems[b])
                if c + 2 < SUP:
                    b2 = (c + 2) % 3
                    pltpu.async_copy(x_hbm.at[idxv.at[c + 2]], rows[b2], sems[b2])
                pltpu.sync_copy(rows[b], acc_sh.at[idxv.at[SUP + c]], add=True)

        plsc.subcore_barrier()

    with jax.named_scope("readout_phase"):
        for k in range(5):
            cno = sid * 5 + k

            @pl.when(cno < NZC)
            def _():
                pltpu.sync_copy(acc_sh.at[pl.ds(cno * CH, CH)], rows_a)
                pltpu.sync_copy(rows_a, out_hbm.at[cid].at[pl.ds(cno * CH, CH)])

        @pl.when(sid == NS - 1)
        def _():
            pltpu.sync_copy(acc_sh.at[pl.ds(NZC * CH, 16)],
                            rows_b.at[pl.ds(0, 16)])
            pltpu.sync_copy(rows_b.at[pl.ds(0, 16)],
                            out_hbm.at[cid].at[pl.ds(NZC * CH, 16)])


_sc_aggregate = functools.partial(
    pl.kernel,
    out_type=jax.ShapeDtypeStruct((NC, N, D), jnp.float32),
    mesh=plsc.VectorSubcoreMesh(core_axis_name="c", subcore_axis_name="s"),
    scratch_types=[
        pltpu.VMEM_SHARED((N, D), jnp.float32),
        pltpu.VMEM((2 * SUP, CH), jnp.int32),
        pltpu.VMEM((CH, D), jnp.float32),
        pltpu.VMEM((CH, D), jnp.float32),
        pltpu.VMEM((CH, D), jnp.float32),
        pltpu.SemaphoreType.DMA,
        pltpu.SemaphoreType.DMA,
        pltpu.SemaphoreType.DMA,
    ],
)(_sc_body)


def _tc_body(aggp_ref, x_ref, bid_ref, wn_ref, ws_ref, b_ref, out_ref):
    i = pl.program_id(0)

    @pl.when(i == 0)
    def _():
        out_ref[...] = jnp.zeros_like(out_ref)

    agg = aggp_ref[0] + aggp_ref[1]
    h = jnp.dot(agg, wn_ref[...], preferred_element_type=jnp.float32)
    h += jnp.dot(x_ref[...], ws_ref[...], preferred_element_type=jnp.float32)
    h = jnp.maximum(h + b_ref[...], 0.0)
    bid = bid_ref[0, 0, :]
    gids = lax.broadcasted_iota(jnp.int32, (G, BN), 0)
    onehot = (gids == bid[None, :]).astype(jnp.float32)
    out_ref[...] += jnp.dot(onehot, h, preferred_element_type=jnp.float32)


def _tc_finish(aggp, x, bids3, w_nbr, w_self, b2):
    return pl.pallas_call(
        _tc_body,
        grid=(NB,),
        in_specs=[
            pl.BlockSpec((NC, BN, D), lambda i: (0, i, 0)),
            pl.BlockSpec((BN, D), lambda i: (i, 0)),
            pl.BlockSpec((1, 1, BN), lambda i: (i, 0, 0)),
            pl.BlockSpec((D, D), lambda i: (0, 0)),
            pl.BlockSpec((D, D), lambda i: (0, 0)),
            pl.BlockSpec((1, D), lambda i: (0, 0)),
        ],
        out_specs=pl.BlockSpec((G, D), lambda i: (0, 0)),
        out_shape=jax.ShapeDtypeStruct((G, D), jnp.float32),
    )(aggp, x, bids3, w_nbr, w_self, b2)


def kernel(x, edge_index, batch_ids, W_nbr, W_self, b):
    E = edge_index.shape[1]
    xz = jnp.concatenate([x, jnp.zeros((1, D), jnp.float32)], axis=0)
    src3 = jnp.pad(edge_index[0], (0, EP - E),
                   constant_values=N).reshape(TSUP, SUP, CH)
    dst3 = jnp.pad(edge_index[1], (0, EP - E)).reshape(TSUP, SUP, CH)
    eidx = jnp.concatenate([src3, dst3], axis=1).reshape(TSUP * 2 * SUP, CH)
    zeros = jnp.zeros((CH, D), jnp.float32)
    aggp = _sc_aggregate(xz, eidx, zeros)
    bids3 = batch_ids.reshape(NB, 1, BN)
    return _tc_finish(aggp, x, bids3, W_nbr, W_self, b.reshape(1, D))
